# baseline (speedup 1.0000x reference)
import math

import jax
import jax.numpy as jnp
from jax.experimental import pallas as pl

_HID = 64
_NL = 2
_ALPHA = 0.1
_LAMDA = 1.0
_EPS = 1e-5
_F32 = jnp.float32
_BF16 = jnp.bfloat16
_PREC = jax.lax.Precision.HIGHEST


def _prep_kernel(x_ref, w_ref, b_ref, adj_ref, h_ref, dinv_ref, adj8_ref):
    x = x_ref[0]
    h = jax.lax.dot_general(x, w_ref[...], (((1,), (1,)), ((), ())),
                            precision=_PREC, preferred_element_type=_F32)
    h_ref[0] = h + b_ref[...]
    a32 = adj_ref[0]
    a = a32.astype(_BF16)
    adj8_ref[0] = a32.astype(jnp.int8)
    ones = jnp.ones((a.shape[0], 1), _BF16)
    deg = jax.lax.dot_general(a, ones, (((0,), (0,)), ((), ())),
                              preferred_element_type=_F32)
    dinv_ref[0] = jax.lax.rsqrt(deg + 1.0)


def _ln_relu(h, g, b):
    mu = jnp.mean(h, axis=1, keepdims=True)
    xc = h - mu
    var = jnp.mean(xc * xc, axis=1, keepdims=True)
    t = xc * jax.lax.rsqrt(var + _EPS) * g + b
    return jnp.maximum(t, 0.0)


def _gcn2(a, t, dinv, w, beta):
    gm = dinv * t
    agg = jax.lax.dot_general(a, gm.astype(_BF16), (((0,), (0,)), ((), ())),
                              preferred_element_type=_F32)
    xx = (1.0 - _ALPHA) * (dinv * (agg + gm)) + _ALPHA * t
    return (1.0 - beta) * xx + beta * jax.lax.dot_general(
        xx, w, (((1,), (0,)), ((), ())),
        precision=_PREC, preferred_element_type=_F32)


def _make_layer1_kernel(beta):
    def _k(h_ref, adj8_ref, dinv_ref, g_ref, b_ref, w_ref, hout_ref):
        a = adj8_ref[0].astype(_BF16)
        h = h_ref[0]
        t = _ln_relu(h, g_ref[...], b_ref[...])
        hout_ref[0] = h + _gcn2(a, t, dinv_ref[0], w_ref[...], beta)
    return _k


def _make_layer2_pred_kernel(beta):
    def _k(h_ref, adj8_ref, dinv_ref, g_ref, b_ref, w_ref, pw_ref, pb_ref,
           o_ref):
        a = adj8_ref[0].astype(_BF16)
        h = h_ref[0]
        t = _ln_relu(h, g_ref[...], b_ref[...])
        hf = h + _gcn2(a, t, dinv_ref[0], w_ref[...], beta)
        o = jnp.sum(hf * pw_ref[...], axis=1, keepdims=True)
        o_ref[0] = o + pb_ref[0, 0]
    return _k


def kernel(x, adj, proj_W, proj_b, ln_g, ln_b, conv_W, pred_W, pred_b):
    bs, N, D = x.shape
    BC = 512
    beta1 = math.log(_LAMDA / 1.0 + 1.0)
    beta2 = math.log(_LAMDA / 2.0 + 1.0)

    h, dinv, adj8 = pl.pallas_call(
        _prep_kernel,
        grid=(bs, N // BC),
        in_specs=[
            pl.BlockSpec((1, BC, D), lambda b, i: (b, i, 0)),
            pl.BlockSpec((_HID, D), lambda b, i: (0, 0)),
            pl.BlockSpec((1, _HID), lambda b, i: (0, 0)),
            pl.BlockSpec((1, N, BC), lambda b, i: (b, 0, i)),
        ],
        out_specs=[
            pl.BlockSpec((1, BC, _HID), lambda b, i: (b, i, 0)),
            pl.BlockSpec((1, BC, 1), lambda b, i: (b, i, 0)),
            pl.BlockSpec((1, N, BC), lambda b, i: (b, 0, i)),
        ],
        out_shape=[
            jax.ShapeDtypeStruct((bs, N, _HID), _F32),
            jax.ShapeDtypeStruct((bs, N, 1), _F32),
            jax.ShapeDtypeStruct((bs, N, N), jnp.int8),
        ],
    )(x, proj_W, proj_b.reshape(1, _HID), adj)

    h = pl.pallas_call(
        _make_layer1_kernel(beta1),
        grid=(bs,),
        in_specs=[
            pl.BlockSpec((1, N, _HID), lambda b: (b, 0, 0)),
            pl.BlockSpec((1, N, N), lambda b: (b, 0, 0)),
            pl.BlockSpec((1, N, 1), lambda b: (b, 0, 0)),
            pl.BlockSpec((1, _HID), lambda b: (0, 0)),
            pl.BlockSpec((1, _HID), lambda b: (0, 0)),
            pl.BlockSpec((_HID, _HID), lambda b: (0, 0)),
        ],
        out_specs=pl.BlockSpec((1, N, _HID), lambda b: (b, 0, 0)),
        out_shape=jax.ShapeDtypeStruct((bs, N, _HID), _F32),
    )(h, adj8, dinv, ln_g[0:1], ln_b[0:1], conv_W[0])

    out = pl.pallas_call(
        _make_layer2_pred_kernel(beta2),
        grid=(bs,),
        in_specs=[
            pl.BlockSpec((1, N, _HID), lambda b: (b, 0, 0)),
            pl.BlockSpec((1, N, N), lambda b: (b, 0, 0)),
            pl.BlockSpec((1, N, 1), lambda b: (b, 0, 0)),
            pl.BlockSpec((1, _HID), lambda b: (0, 0)),
            pl.BlockSpec((1, _HID), lambda b: (0, 0)),
            pl.BlockSpec((_HID, _HID), lambda b: (0, 0)),
            pl.BlockSpec((1, _HID), lambda b: (0, 0)),
            pl.BlockSpec((1, 1), lambda b: (0, 0)),
        ],
        out_specs=pl.BlockSpec((1, N, 1), lambda b: (b, 0, 0)),
        out_shape=jax.ShapeDtypeStruct((bs, N, 1), _F32),
    )(h, adj8, dinv, ln_g[1:2], ln_b[1:2], conv_W[1], pred_W,
      pred_b.reshape(1, 1))
    return out


# speedup vs baseline: 1.3538x; 1.3538x over previous
import math

import jax
import jax.numpy as jnp
from jax.experimental import pallas as pl

_HID = 64
_NL = 2
_ALPHA = 0.1
_LAMDA = 1.0
_EPS = 1e-5
_F32 = jnp.float32
_BF16 = jnp.bfloat16
_PREC = jax.lax.Precision.HIGHEST


def _proj_kernel(x_ref, w_ref, b_ref, h_ref):
    x = x_ref[0]
    h = jax.lax.dot_general(x, w_ref[...], (((1,), (1,)), ((), ())),
                            precision=_PREC, preferred_element_type=_F32)
    h_ref[0] = h + b_ref[...]


def _ln_relu(h, g, b):
    mu = jnp.mean(h, axis=1, keepdims=True)
    xc = h - mu
    var = jnp.mean(xc * xc, axis=1, keepdims=True)
    t = xc * jax.lax.rsqrt(var + _EPS) * g + b
    return jnp.maximum(t, 0.0)


def _gcn2(a, t, dinv, w, beta):
    gm = dinv * t
    gm16t = jnp.transpose(gm.astype(_BF16))
    aggt = jax.lax.dot_general(gm16t, a, (((1,), (0,)), ((), ())),
                               preferred_element_type=_F32)
    agg = jnp.transpose(aggt)
    xx = (1.0 - _ALPHA) * (dinv * (agg + gm)) + _ALPHA * t
    return (1.0 - beta) * xx + beta * jax.lax.dot_general(
        xx, w, (((1,), (0,)), ((), ())),
        precision=_PREC, preferred_element_type=_F32)


def _make_layer1_kernel(beta):
    def _k(h_ref, adj_ref, g_ref, b_ref, w_ref, hout_ref, adj8_ref, dinv_ref):
        a32 = adj_ref[0]
        a = a32.astype(_BF16)
        adj8_ref[0] = a32.astype(jnp.int8)
        ones = jnp.ones((1, a.shape[0]), _BF16)
        deg = jax.lax.dot_general(ones, a, (((1,), (0,)), ((), ())),
                                  preferred_element_type=_F32)
        dinv = jnp.transpose(jax.lax.rsqrt(deg + 1.0))
        dinv_ref[0] = dinv
        h = h_ref[0]
        t = _ln_relu(h, g_ref[...], b_ref[...])
        hout_ref[0] = h + _gcn2(a, t, dinv, w_ref[...], beta)
    return _k


def _make_layer2_pred_kernel(beta):
    def _k(h_ref, adj8_ref, dinv_ref, g_ref, b_ref, w_ref, pw_ref, pb_ref,
           o_ref):
        a = adj8_ref[0].astype(_BF16)
        h = h_ref[0]
        t = _ln_relu(h, g_ref[...], b_ref[...])
        hf = h + _gcn2(a, t, dinv_ref[0], w_ref[...], beta)
        o = jnp.sum(hf * pw_ref[...], axis=1, keepdims=True)
        o_ref[0] = o + pb_ref[0, 0]
    return _k


def kernel(x, adj, proj_W, proj_b, ln_g, ln_b, conv_W, pred_W, pred_b):
    bs, N, D = x.shape
    beta1 = math.log(_LAMDA / 1.0 + 1.0)
    beta2 = math.log(_LAMDA / 2.0 + 1.0)

    h = pl.pallas_call(
        _proj_kernel,
        grid=(bs,),
        in_specs=[
            pl.BlockSpec((1, N, D), lambda b: (b, 0, 0)),
            pl.BlockSpec((_HID, D), lambda b: (0, 0)),
            pl.BlockSpec((1, _HID), lambda b: (0, 0)),
        ],
        out_specs=pl.BlockSpec((1, N, _HID), lambda b: (b, 0, 0)),
        out_shape=jax.ShapeDtypeStruct((bs, N, _HID), _F32),
    )(x, proj_W, proj_b.reshape(1, _HID))

    h, adj8, dinv = pl.pallas_call(
        _make_layer1_kernel(beta1),
        grid=(bs,),
        in_specs=[
            pl.BlockSpec((1, N, _HID), lambda b: (b, 0, 0)),
            pl.BlockSpec((1, N, N), lambda b: (b, 0, 0)),
            pl.BlockSpec((1, _HID), lambda b: (0, 0)),
            pl.BlockSpec((1, _HID), lambda b: (0, 0)),
            pl.BlockSpec((_HID, _HID), lambda b: (0, 0)),
        ],
        out_specs=[
            pl.BlockSpec((1, N, _HID), lambda b: (b, 0, 0)),
            pl.BlockSpec((1, N, N), lambda b: (b, 0, 0)),
            pl.BlockSpec((1, N, 1), lambda b: (b, 0, 0)),
        ],
        out_shape=[
            jax.ShapeDtypeStruct((bs, N, _HID), _F32),
            jax.ShapeDtypeStruct((bs, N, N), jnp.int8),
            jax.ShapeDtypeStruct((bs, N, 1), _F32),
        ],
    )(h, adj, ln_g[0:1], ln_b[0:1], conv_W[0])

    out = pl.pallas_call(
        _make_layer2_pred_kernel(beta2),
        grid=(bs,),
        in_specs=[
            pl.BlockSpec((1, N, _HID), lambda b: (b, 0, 0)),
            pl.BlockSpec((1, N, N), lambda b: (b, 0, 0)),
            pl.BlockSpec((1, N, 1), lambda b: (b, 0, 0)),
            pl.BlockSpec((1, _HID), lambda b: (0, 0)),
            pl.BlockSpec((1, _HID), lambda b: (0, 0)),
            pl.BlockSpec((_HID, _HID), lambda b: (0, 0)),
            pl.BlockSpec((1, _HID), lambda b: (0, 0)),
            pl.BlockSpec((1, 1), lambda b: (0, 0)),
        ],
        out_specs=pl.BlockSpec((1, N, 1), lambda b: (b, 0, 0)),
        out_shape=jax.ShapeDtypeStruct((bs, N, 1), _F32),
    )(h, adj8, dinv, ln_g[1:2], ln_b[1:2], conv_W[1], pred_W,
      pred_b.reshape(1, 1))
    return out


# speedup vs baseline: 1.6766x; 1.2384x over previous
import math

import jax
import jax.numpy as jnp
from jax.experimental import pallas as pl

_HID = 64
_NL = 2
_ALPHA = 0.1
_LAMDA = 1.0
_EPS = 1e-5
_F32 = jnp.float32
_BF16 = jnp.bfloat16
_PREC = jax.lax.Precision.HIGHEST


def _ln_relu(h, g, b):
    mu = jnp.mean(h, axis=1, keepdims=True)
    xc = h - mu
    var = jnp.mean(xc * xc, axis=1, keepdims=True)
    t = xc * jax.lax.rsqrt(var + _EPS) * g + b
    return jnp.maximum(t, 0.0)


def _gcn2(a, t, dinv, w, beta):
    gm = dinv * t
    gm16t = jnp.transpose(gm.astype(_BF16))
    aggt = jax.lax.dot_general(gm16t, a, (((1,), (0,)), ((), ())),
                               preferred_element_type=_F32)
    agg = jnp.transpose(aggt)
    xx = (1.0 - _ALPHA) * (dinv * (agg + gm)) + _ALPHA * t
    return (1.0 - beta) * xx + beta * jax.lax.dot_general(
        xx, w, (((1,), (0,)), ((), ())),
        precision=_PREC, preferred_element_type=_F32)


def _fused_kernel(x_ref, adj_ref, pw_ref, pb_ref, g_ref, b_ref, w0_ref,
                  w1_ref, ow_ref, ob_ref, o_ref):
    beta1 = math.log(_LAMDA / 1.0 + 1.0)
    beta2 = math.log(_LAMDA / 2.0 + 1.0)
    h = jax.lax.dot_general(x_ref[0], pw_ref[...], (((1,), (1,)), ((), ())),
                            precision=_PREC, preferred_element_type=_F32)
    h = h + pb_ref[...]
    a = adj_ref[0].astype(_BF16)
    ones = jnp.ones((1, a.shape[0]), _BF16)
    deg = jax.lax.dot_general(ones, a, (((1,), (0,)), ((), ())),
                              preferred_element_type=_F32)
    dinv = jnp.transpose(jax.lax.rsqrt(deg + 1.0))
    t = _ln_relu(h, g_ref[0:1, :], b_ref[0:1, :])
    h = h + _gcn2(a, t, dinv, w0_ref[...], beta1)
    t = _ln_relu(h, g_ref[1:2, :], b_ref[1:2, :])
    h = h + _gcn2(a, t, dinv, w1_ref[...], beta2)
    o = jnp.sum(h * ow_ref[...], axis=1, keepdims=True)
    o_ref[0] = o + ob_ref[0, 0]


def kernel(x, adj, proj_W, proj_b, ln_g, ln_b, conv_W, pred_W, pred_b):
    bs, N, D = x.shape
    return pl.pallas_call(
        _fused_kernel,
        grid=(bs,),
        in_specs=[
            pl.BlockSpec((1, N, D), lambda b: (b, 0, 0)),
            pl.BlockSpec((1, N, N), lambda b: (b, 0, 0)),
            pl.BlockSpec((_HID, D), lambda b: (0, 0)),
            pl.BlockSpec((1, _HID), lambda b: (0, 0)),
            pl.BlockSpec((_NL, _HID), lambda b: (0, 0)),
            pl.BlockSpec((_NL, _HID), lambda b: (0, 0)),
            pl.BlockSpec((_HID, _HID), lambda b: (0, 0)),
            pl.BlockSpec((_HID, _HID), lambda b: (0, 0)),
            pl.BlockSpec((1, _HID), lambda b: (0, 0)),
            pl.BlockSpec((1, 1), lambda b: (0, 0)),
        ],
        out_specs=pl.BlockSpec((1, N, 1), lambda b: (b, 0, 0)),
        out_shape=jax.ShapeDtypeStruct((bs, N, 1), _F32),
    )(x, adj, proj_W, proj_b.reshape(1, _HID), ln_g, ln_b, conv_W[0],
      conv_W[1], pred_W, pred_b.reshape(1, 1))
